# baseline (speedup 1.0000x reference)
import functools

import jax
import jax.numpy as jnp
from jax import lax
from jax.experimental import pallas as pl
from jax.experimental.pallas import tpu as pltpu
from jax.experimental.pallas import tpu_sc as plsc

_P = jax.lax.Precision.HIGHEST

N_NODES = 50000
N_PAD = 50176
HALF = 32
GROUP = 128
G_INFLIGHT = 4
G_IDX = 8
GROUPS_PER_TILE = 400
E_PAD = 16 * GROUPS_PER_TILE * GROUP
N_SUPER = GROUPS_PER_TILE // G_IDX
ROWS_PER_TILE = N_PAD // 16
STAGE_ROWS = 98
N_STAGE = ROWS_PER_TILE // STAGE_ROWS


def _make_segsum(with_deg):
  mesh = plsc.VectorSubcoreMesh(core_axis_name="c", subcore_axis_name="s")
  out_type = [jax.ShapeDtypeStruct((2 * N_PAD, HALF), jnp.float32)]
  if with_deg:
    out_type.append(jax.ShapeDtypeStruct((N_PAD,), jnp.float32))
  scratch = [
      pltpu.VMEM((2, G_IDX, GROUP), jnp.int32),
      pltpu.VMEM((2, G_IDX, GROUP), jnp.int32),
      pltpu.VMEM((2, 2, GROUP, HALF), jnp.float32),
      pltpu.VMEM((STAGE_ROWS, HALF), jnp.float32),
      pltpu.VMEM((ROWS_PER_TILE,), jnp.float32),
      pltpu.VMEM((GROUP,), jnp.float32),
      pltpu.VMEM_SHARED((N_PAD, HALF), jnp.float32),
      pltpu.VMEM_SHARED((N_PAD,), jnp.float32),
      pltpu.SemaphoreType.DMA,
      pltpu.SemaphoreType.DMA,
      pltpu.SemaphoreType.DMA,
      pltpu.SemaphoreType.DMA,
  ]

  def body(h_flat, srcp, dstp, *rest):
    if with_deg:
      msum_out, deg_out = rest[0], rest[1]
      rest = rest[2:]
    else:
      msum_out, deg_out = rest[0], None
      rest = rest[1:]
    (srcbuf, dstbuf, rows, stage, dstage, ones, acc, accd,
     gsem, ssem, dsem, isem) = rest

    c = lax.axis_index("c")
    s = lax.axis_index("s")
    base = s * ROWS_PER_TILE
    gbase = s * GROUPS_PER_TILE
    off = c * N_PAD

    zv = jnp.zeros((16,), jnp.float32)

    def _zrow(i, carry):
      stage[i, 0:16] = zv
      stage[i, 16:32] = zv
      return carry

    lax.fori_loop(0, STAGE_ROWS, _zrow, 0)

    def _zacc(j, carry):
      pltpu.sync_copy(stage, acc.at[pl.ds(base + j * STAGE_ROWS, STAGE_ROWS)])
      return carry

    lax.fori_loop(0, N_STAGE, _zacc, 0)
    if with_deg:
      @pl.when(c == 0)
      def _deg_init():
        def _zd(i, carry):
          dstage[pl.ds(i * 16, 16)] = zv
          return carry
        lax.fori_loop(0, ROWS_PER_TILE // 16, _zd, 0)
        pltpu.sync_copy(dstage, accd.at[pl.ds(base, ROWS_PER_TILE)])
        ov = jnp.ones((16,), jnp.float32)
        for g in range(GROUP // 16):
          ones[pl.ds(g * 16, 16)] = ov
    plsc.subcore_barrier()

    def idx_issue(i_sc, q):
      pltpu.async_copy(srcp.at[pl.ds(gbase + i_sc * G_IDX, G_IDX)],
                       srcbuf.at[q], isem)
      pltpu.async_copy(dstp.at[pl.ds(gbase + i_sc * G_IDX, G_IDX)],
                       dstbuf.at[q], isem)

    def idx_wait(q):
      for _ in range(2):
        pltpu.make_async_copy(srcp.at[pl.ds(gbase, G_IDX)],
                              srcbuf.at[q], isem).wait()

    def add_off(q):
      for g in range(G_IDX):
        def _addoff(j, carry2, g=g):
          srcbuf[q, g, pl.ds(j * 16, 16)] = (
              srcbuf[q, g, pl.ds(j * 16, 16)] + off)
          return carry2
        lax.fori_loop(0, GROUP // 16, _addoff, 0)

    def g_issue(q, k, b):
      for g2 in range(2):
        pltpu.async_copy(h_flat.at[srcbuf.at[q, 2 * k + g2]],
                         rows.at[b, g2], gsem)

    def g_wait():
      for g2 in range(2):
        pltpu.make_async_copy(h_flat.at[pl.ds(0, GROUP)],
                              rows.at[0, g2], gsem).wait()

    def s_issue(q, k, b):
      for g2 in range(2):
        pltpu.async_copy(rows.at[b, g2], acc.at[dstbuf.at[q, 2 * k + g2]],
                         ssem, add=True)
      if with_deg:
        @pl.when(c == 0)
        def _():
          for g2 in range(2):
            pltpu.async_copy(ones, accd.at[dstbuf.at[q, 2 * k + g2]], dsem,
                             add=True)

    def s_wait():
      for g2 in range(2):
        pltpu.make_async_copy(h_flat.at[pl.ds(0, GROUP)],
                              acc.at[pl.ds(0, GROUP)], ssem).wait()

    def drain_prev():
      s_wait()
      s_wait()
      if with_deg:
        @pl.when(c == 0)
        def _():
          pltpu.make_async_copy(deg_out.at[pl.ds(0, 1024)],
                                accd.at[pl.ds(0, 1024)], dsem).wait()

    def proc(q, notfirst, have_next, i_sc):
      idx_wait(q)
      if notfirst is None:
        drain_prev()
      else:
        @pl.when(notfirst)
        def _():
          drain_prev()
      if have_next is None:
        idx_issue(i_sc + 1, 1 - q)
      else:
        @pl.when(have_next)
        def _():
          idx_issue(i_sc + 1, 1 - q)
      add_off(q)
      g_issue(q, 0, 0)
      g_issue(q, 1, 1)
      g_wait()
      s_issue(q, 0, 0)
      g_wait()
      s_issue(q, 1, 1)
      s_wait()
      g_issue(q, 2, 0)
      s_wait()
      g_issue(q, 3, 1)
      g_wait()
      s_issue(q, 2, 0)
      g_wait()
      s_issue(q, 3, 1)

    idx_issue(0, 0)

    def super2(i2, carry):
      i_sc0 = 2 * i2
      proc(0, i_sc0 > 0, None, i_sc0)
      proc(1, None, i2 < (N_SUPER // 2 - 1), i_sc0 + 1)
      return carry

    lax.fori_loop(0, N_SUPER // 2, super2, 0)
    drain_prev()
    plsc.subcore_barrier()

    def _out(j, carry):
      pltpu.sync_copy(acc.at[pl.ds(base + j * STAGE_ROWS, STAGE_ROWS)], stage)
      pltpu.sync_copy(
          stage,
          msum_out.at[pl.ds(off + base + j * STAGE_ROWS, STAGE_ROWS)])
      return carry

    lax.fori_loop(0, N_STAGE, _out, 0)
    if with_deg:
      @pl.when(c == 0)
      def _deg_out():
        pltpu.sync_copy(accd.at[pl.ds(base, ROWS_PER_TILE)], dstage)
        pltpu.sync_copy(dstage, deg_out.at[pl.ds(base, ROWS_PER_TILE)])

  return pl.kernel(body, out_type=tuple(out_type), mesh=mesh,
                   scratch_types=scratch,
                   compiler_params=pltpu.CompilerParams(
                       use_tc_tiling_on_sc=False))


_R = 1792
_GRID = N_PAD // _R


def _proj_body(x_ref, t_ref, w1_ref, b1_ref, w2_ref, b2_ref, out_ref):
  x = x_ref[...]
  t = t_ref[...]
  h1 = jnp.dot(x, w1_ref[...]) + b1_ref[...]
  h1 = jnp.maximum(h1, 0.0)
  tcol = lax.broadcasted_iota(jnp.int32, (1, 192), 1) // 64
  h1m = jnp.where(t == tcol, h1, 0.0)
  oneh = (t == lax.broadcasted_iota(jnp.int32, (1, 3), 1)).astype(jnp.float32)
  out_ref[...] = jnp.dot(h1m, w2_ref[0]) + jnp.dot(oneh, b2_ref[0])


def _sage_body(relu, hl_ref, hh_ref, ml_ref, mh_ref, d_ref,
               ws_ref, wn_ref, b_ref, out_ref):
  h = jnp.concatenate([hl_ref[...], hh_ref[...]], axis=1)
  m = jnp.concatenate([ml_ref[...], mh_ref[...]], axis=1)
  deg = jnp.maximum(d_ref[...], 1.0)
  hn = m / deg
  o = jnp.dot(h, ws_ref[0]) + jnp.dot(hn, wn_ref[0]) + b_ref[0]
  if relu:
    o = jnp.maximum(o, 0.0)
  out_ref[...] = o


def _final_body(hl_ref, hh_ref, ml_ref, mh_ref, d_ref, ws_ref, wn_ref, b_ref,
                cw0, cb0, cw1, cb1, cw2, cb2, cw3, cb3,
                rw0, rb0, rw1, rb1, rw2, rb2, rw3, rb3,
                c_out, r_out):
  h = jnp.concatenate([hl_ref[...], hh_ref[...]], axis=1)
  m = jnp.concatenate([ml_ref[...], mh_ref[...]], axis=1)
  deg = jnp.maximum(d_ref[...], 1.0)
  hn = m / deg
  h2 = jnp.dot(h, ws_ref[...]) + jnp.dot(hn, wn_ref[...]) + b_ref[...]
  cc = h2
  for i, (w, b) in enumerate([(cw0, cb0), (cw1, cb1), (cw2, cb2), (cw3, cb3)]):
    cc = jnp.dot(cc, w[...]) + b[...]
    if i != 3:
      cc = jnp.maximum(cc, 0.0)
  rr = h2
  for i, (w, b) in enumerate([(rw0, rb0), (rw1, rb1), (rw2, rb2),
                              (rw3, rb3)]):
    rr = jnp.dot(rr, w[...]) + b[...]
    if i != 3:
      rr = jnp.maximum(rr, 0.0)
  c_out[...] = cc
  r_out[...] = rr


def _full(shape):
  return pl.BlockSpec(shape, lambda *a: tuple(0 for _ in shape))


def _call_proj(xp, tp, w1, b1, w2, b2):
  return pl.pallas_call(
      _proj_body,
      grid=(2, _GRID),
      in_specs=[
          pl.BlockSpec((_R, 16), lambda j, i: (i, 0)),
          pl.BlockSpec((_R, 1), lambda j, i: (i, 0)),
          pl.BlockSpec((16, 192), lambda j, i: (0, 0)),
          pl.BlockSpec((1, 192), lambda j, i: (0, 0)),
          pl.BlockSpec((1, 192, HALF), lambda j, i: (j, 0, 0)),
          pl.BlockSpec((1, 3, HALF), lambda j, i: (j, 0, 0)),
      ],
      out_specs=pl.BlockSpec((_R, HALF), lambda j, i: (j * _GRID + i, 0)),
      out_shape=jax.ShapeDtypeStruct((2 * N_PAD, HALF), jnp.float32),
  )(xp, tp, w1, b1, w2, b2)


def _call_sage(relu, h, msum, deg, ws, wn, b):
  return pl.pallas_call(
      functools.partial(_sage_body, relu),
      grid=(2, _GRID),
      in_specs=[
          pl.BlockSpec((_R, HALF), lambda j, i: (i, 0)),
          pl.BlockSpec((_R, HALF), lambda j, i: (_GRID + i, 0)),
          pl.BlockSpec((_R, HALF), lambda j, i: (i, 0)),
          pl.BlockSpec((_R, HALF), lambda j, i: (_GRID + i, 0)),
          pl.BlockSpec((_R, 1), lambda j, i: (i, 0)),
          pl.BlockSpec((1, 64, HALF), lambda j, i: (j, 0, 0)),
          pl.BlockSpec((1, 64, HALF), lambda j, i: (j, 0, 0)),
          pl.BlockSpec((1, 1, HALF), lambda j, i: (j, 0, 0)),
      ],
      out_specs=pl.BlockSpec((_R, HALF), lambda j, i: (j * _GRID + i, 0)),
      out_shape=jax.ShapeDtypeStruct((2 * N_PAD, HALF), jnp.float32),
  )(h, h, msum, msum, deg, ws, wn, b)


def _call_final(h, msum, deg, ws, wn, b, cws, rws):
  in_specs = [
      pl.BlockSpec((_R, HALF), lambda i: (i, 0)),
      pl.BlockSpec((_R, HALF), lambda i: (_GRID + i, 0)),
      pl.BlockSpec((_R, HALF), lambda i: (i, 0)),
      pl.BlockSpec((_R, HALF), lambda i: (_GRID + i, 0)),
      pl.BlockSpec((_R, 1), lambda i: (i, 0)),
      _full((64, 64)), _full((64, 64)), _full((1, 64)),
  ]
  args = [h, h, msum, msum, deg, ws, wn, b]
  for (w, bb) in cws:
    in_specs += [_full(w.shape), _full(bb.shape)]
    args += [w, bb]
  for (w, bb) in rws:
    in_specs += [_full(w.shape), _full(bb.shape)]
    args += [w, bb]
  return pl.pallas_call(
      _final_body,
      grid=(_GRID,),
      in_specs=in_specs,
      out_specs=[pl.BlockSpec((_R, 8), lambda i: (i, 0)),
                 pl.BlockSpec((_R, 1), lambda i: (i, 0))],
      out_shape=[jax.ShapeDtypeStruct((N_NODES, 8), jnp.float32),
                 jax.ShapeDtypeStruct((N_NODES, 1), jnp.float32)],
  )(*args)


def kernel(feats, params, ntypes, edge_index, dim_list):
  n = feats.shape[0]
  e = edge_index.shape[1]

  tp = ntypes.astype(jnp.int32)[:, None]

  src = edge_index[0].astype(jnp.int32).reshape(e // GROUP, GROUP)
  dst = edge_index[1].astype(jnp.int32).reshape(e // GROUP, GROUP)
  npad_g = E_PAD // GROUP - e // GROUP
  ar = jnp.arange(npad_g * GROUP, dtype=jnp.int32)
  pad_src = (ar % n).reshape(npad_g, GROUP)
  pad_dst = (n + (ar % 128)).reshape(npad_g, GROUP)
  srcp = jnp.concatenate([src, pad_src], axis=0)
  dstp = jnp.concatenate([dst, pad_dst], axis=0)

  def halves(w):
    return jnp.stack([w[..., :HALF], w[..., HALF:]], axis=0)

  pj = params["proj"]
  w1 = jnp.concatenate([pj[t]["l1"]["W"] for t in range(3)], axis=1)
  b1 = jnp.concatenate([pj[t]["l1"]["b"] for t in range(3)])[None, :]
  w2 = halves(jnp.concatenate([pj[t]["l2"]["W"] for t in range(3)], axis=0))
  b2 = halves(jnp.stack([pj[t]["l2"]["b"] for t in range(3)], axis=0))
  sg = params["sage"]
  cws = [(p["W"], p["b"][None, :]) for p in params["cmlp"]]
  rws = [(p["W"], p["b"][None, :]) for p in params["reg"]]

  segsum_deg = _make_segsum(True)
  segsum = _make_segsum(False)

  h0 = _call_proj(feats[:, :16], tp, w1, b1, w2, b2)
  msum0, deg2 = segsum_deg(h0, srcp, dstp)
  deg = deg2[:, None]
  h1 = _call_sage(True, h0, msum0, deg,
                  halves(sg[0]["Wself"]), halves(sg[0]["Wneigh"]),
                  halves(sg[0]["b"][None, :]))
  msum1 = segsum(h1, srcp, dstp)
  if isinstance(msum1, (tuple, list)):
    msum1 = msum1[0]
  cc, rr = _call_final(h1, msum1, deg,
                       sg[1]["Wself"], sg[1]["Wneigh"], sg[1]["b"][None, :],
                       cws, rws)
  return cc, rr


# speedup vs baseline: 1.0032x; 1.0032x over previous
import functools

import jax
import jax.numpy as jnp
from jax import lax
from jax.experimental import pallas as pl
from jax.experimental.pallas import tpu as pltpu
from jax.experimental.pallas import tpu_sc as plsc

N_NODES = 50000
N_PAD = 50176
HALF = 32
GROUP = 128
G_INFLIGHT = 4
G_IDX = 8
GROUPS_PER_TILE = 400
E_PAD = 16 * GROUPS_PER_TILE * GROUP
N_SUPER = GROUPS_PER_TILE // G_IDX
ROWS_PER_TILE = N_PAD // 16
STAGE_ROWS = 98
N_STAGE = ROWS_PER_TILE // STAGE_ROWS


def _make_segsum(with_deg):
  mesh = plsc.VectorSubcoreMesh(core_axis_name="c", subcore_axis_name="s")
  out_type = [jax.ShapeDtypeStruct((2 * N_PAD, HALF), jnp.float32)]
  if with_deg:
    out_type.append(jax.ShapeDtypeStruct((N_PAD,), jnp.float32))
  scratch = [
      pltpu.VMEM((2, G_IDX, GROUP), jnp.int32),
      pltpu.VMEM((2, G_IDX, GROUP), jnp.int32),
      pltpu.VMEM((2, 2, GROUP, HALF), jnp.float32),
      pltpu.VMEM((STAGE_ROWS, HALF), jnp.float32),
      pltpu.VMEM((ROWS_PER_TILE,), jnp.float32),
      pltpu.VMEM((GROUP,), jnp.float32),
      pltpu.VMEM_SHARED((N_PAD, HALF), jnp.float32),
      pltpu.VMEM_SHARED((N_PAD,), jnp.float32),
      pltpu.SemaphoreType.DMA,
      pltpu.SemaphoreType.DMA,
      pltpu.SemaphoreType.DMA,
      pltpu.SemaphoreType.DMA,
  ]

  def body(h_flat, srcp, dstp, *rest):
    if with_deg:
      msum_out, deg_out = rest[0], rest[1]
      rest = rest[2:]
    else:
      msum_out, deg_out = rest[0], None
      rest = rest[1:]
    (srcbuf, dstbuf, rows, stage, dstage, ones, acc, accd,
     gsem, ssem, dsem, isem) = rest

    c = lax.axis_index("c")
    s = lax.axis_index("s")
    base = s * ROWS_PER_TILE
    gbase = s * GROUPS_PER_TILE
    off = c * N_PAD

    zv = jnp.zeros((16,), jnp.float32)

    def _zrow(i, carry):
      stage[i, 0:16] = zv
      stage[i, 16:32] = zv
      return carry

    lax.fori_loop(0, STAGE_ROWS, _zrow, 0)

    def _zacc(j, carry):
      pltpu.sync_copy(stage, acc.at[pl.ds(base + j * STAGE_ROWS, STAGE_ROWS)])
      return carry

    lax.fori_loop(0, N_STAGE, _zacc, 0)
    if with_deg:
      @pl.when(c == 0)
      def _deg_init():
        def _zd(i, carry):
          dstage[pl.ds(i * 16, 16)] = zv
          return carry
        lax.fori_loop(0, ROWS_PER_TILE // 16, _zd, 0)
        pltpu.sync_copy(dstage, accd.at[pl.ds(base, ROWS_PER_TILE)])
        ov = jnp.ones((16,), jnp.float32)
        for g in range(GROUP // 16):
          ones[pl.ds(g * 16, 16)] = ov
    plsc.subcore_barrier()

    def idx_issue(i_sc, q):
      pltpu.async_copy(srcp.at[pl.ds(gbase + i_sc * G_IDX, G_IDX)],
                       srcbuf.at[q], isem)
      pltpu.async_copy(dstp.at[pl.ds(gbase + i_sc * G_IDX, G_IDX)],
                       dstbuf.at[q], isem)

    def idx_wait(q):
      for _ in range(2):
        pltpu.make_async_copy(srcp.at[pl.ds(gbase, G_IDX)],
                              srcbuf.at[q], isem).wait()

    def add_off(q):
      for g in range(G_IDX):
        def _addoff(j, carry2, g=g):
          srcbuf[q, g, pl.ds(j * 16, 16)] = (
              srcbuf[q, g, pl.ds(j * 16, 16)] + off)
          return carry2
        lax.fori_loop(0, GROUP // 16, _addoff, 0)

    def g_issue(q, k, b):
      for g2 in range(2):
        pltpu.async_copy(h_flat.at[srcbuf.at[q, 2 * k + g2]],
                         rows.at[b, g2], gsem)

    def g_wait():
      for g2 in range(2):
        pltpu.make_async_copy(h_flat.at[pl.ds(0, GROUP)],
                              rows.at[0, g2], gsem).wait()

    def s_issue(q, k, b):
      for g2 in range(2):
        pltpu.async_copy(rows.at[b, g2], acc.at[dstbuf.at[q, 2 * k + g2]],
                         ssem, add=True)
      if with_deg:
        @pl.when(c == 0)
        def _():
          for g2 in range(2):
            pltpu.async_copy(ones, accd.at[dstbuf.at[q, 2 * k + g2]], dsem,
                             add=True)

    def s_wait():
      for g2 in range(2):
        pltpu.make_async_copy(h_flat.at[pl.ds(0, GROUP)],
                              acc.at[pl.ds(0, GROUP)], ssem).wait()

    def drain_prev():
      s_wait()
      s_wait()
      if with_deg:
        @pl.when(c == 0)
        def _():
          pltpu.make_async_copy(deg_out.at[pl.ds(0, 1024)],
                                accd.at[pl.ds(0, 1024)], dsem).wait()

    def proc(q, notfirst, have_next, i_sc):
      idx_wait(q)
      if notfirst is None:
        drain_prev()
      else:
        @pl.when(notfirst)
        def _():
          drain_prev()
      if have_next is None:
        idx_issue(i_sc + 1, 1 - q)
      else:
        @pl.when(have_next)
        def _():
          idx_issue(i_sc + 1, 1 - q)
      add_off(q)
      g_issue(q, 0, 0)
      g_issue(q, 1, 1)
      g_wait()
      s_issue(q, 0, 0)
      g_wait()
      s_issue(q, 1, 1)
      s_wait()
      g_issue(q, 2, 0)
      s_wait()
      g_issue(q, 3, 1)
      g_wait()
      s_issue(q, 2, 0)
      g_wait()
      s_issue(q, 3, 1)

    idx_issue(0, 0)

    def super2(i2, carry):
      i_sc0 = 2 * i2
      proc(0, i_sc0 > 0, None, i_sc0)
      proc(1, None, i2 < (N_SUPER // 2 - 1), i_sc0 + 1)
      return carry

    lax.fori_loop(0, N_SUPER // 2, super2, 0)
    drain_prev()
    plsc.subcore_barrier()

    def _out(j, carry):
      pltpu.sync_copy(acc.at[pl.ds(base + j * STAGE_ROWS, STAGE_ROWS)], stage)
      pltpu.sync_copy(
          stage,
          msum_out.at[pl.ds(off + base + j * STAGE_ROWS, STAGE_ROWS)])
      return carry

    lax.fori_loop(0, N_STAGE, _out, 0)
    if with_deg:
      @pl.when(c == 0)
      def _deg_out():
        pltpu.sync_copy(accd.at[pl.ds(base, ROWS_PER_TILE)], dstage)
        pltpu.sync_copy(dstage, deg_out.at[pl.ds(base, ROWS_PER_TILE)])

  return pl.kernel(body, out_type=tuple(out_type), mesh=mesh,
                   scratch_types=scratch,
                   compiler_params=pltpu.CompilerParams(
                       use_tc_tiling_on_sc=False))


_R = 1792
_GRID = N_PAD // _R


def _proj_body(x_ref, t_ref, w1_ref, b1_ref, w2_ref, b2_ref, out_ref):
  x = x_ref[...]
  t = t_ref[...]
  h1 = jnp.dot(x, w1_ref[...]) + b1_ref[...]
  h1 = jnp.maximum(h1, 0.0)
  tcol = lax.broadcasted_iota(jnp.int32, (1, 192), 1) // 64
  h1m = jnp.where(t == tcol, h1, 0.0)
  oneh = (t == lax.broadcasted_iota(jnp.int32, (1, 3), 1)).astype(jnp.float32)
  out_ref[...] = jnp.dot(h1m, w2_ref[0]) + jnp.dot(oneh, b2_ref[0])


def _sage_body(relu, hl_ref, hh_ref, ml_ref, mh_ref, d_ref,
               ws_ref, wn_ref, b_ref, out_ref):
  h = jnp.concatenate([hl_ref[...], hh_ref[...]], axis=1)
  m = jnp.concatenate([ml_ref[...], mh_ref[...]], axis=1)
  deg = jnp.maximum(d_ref[...], 1.0)
  hn = m / deg
  o = jnp.dot(h, ws_ref[0]) + jnp.dot(hn, wn_ref[0]) + b_ref[0]
  if relu:
    o = jnp.maximum(o, 0.0)
  out_ref[...] = o


def _final_body(hl_ref, hh_ref, ml_ref, mh_ref, d_ref, ws_ref, wn_ref, b_ref,
                cw0, cb0, cw1, cb1, cw2, cb2, cw3, cb3,
                rw0, rb0, rw1, rb1, rw2, rb2, rw3, rb3,
                c_out, r_out):
  h = jnp.concatenate([hl_ref[...], hh_ref[...]], axis=1)
  m = jnp.concatenate([ml_ref[...], mh_ref[...]], axis=1)
  deg = jnp.maximum(d_ref[...], 1.0)
  hn = m / deg
  h2 = jnp.dot(h, ws_ref[...]) + jnp.dot(hn, wn_ref[...]) + b_ref[...]
  cc = h2
  for i, (w, b) in enumerate([(cw0, cb0), (cw1, cb1), (cw2, cb2), (cw3, cb3)]):
    cc = jnp.dot(cc, w[...]) + b[...]
    if i != 3:
      cc = jnp.maximum(cc, 0.0)
  rr = h2
  for i, (w, b) in enumerate([(rw0, rb0), (rw1, rb1), (rw2, rb2),
                              (rw3, rb3)]):
    rr = jnp.dot(rr, w[...]) + b[...]
    if i != 3:
      rr = jnp.maximum(rr, 0.0)
  c_out[...] = cc
  r_out[...] = rr


def _full(shape):
  return pl.BlockSpec(shape, lambda *a: tuple(0 for _ in shape))


def _call_proj(xp, tp, w1, b1, w2, b2):
  return pl.pallas_call(
      _proj_body,
      grid=(2, _GRID),
      in_specs=[
          pl.BlockSpec((_R, 16), lambda j, i: (i, 0)),
          pl.BlockSpec((_R, 1), lambda j, i: (i, 0)),
          pl.BlockSpec((16, 192), lambda j, i: (0, 0)),
          pl.BlockSpec((1, 192), lambda j, i: (0, 0)),
          pl.BlockSpec((1, 192, HALF), lambda j, i: (j, 0, 0)),
          pl.BlockSpec((1, 3, HALF), lambda j, i: (j, 0, 0)),
      ],
      out_specs=pl.BlockSpec((_R, HALF), lambda j, i: (j * _GRID + i, 0)),
      out_shape=jax.ShapeDtypeStruct((2 * N_PAD, HALF), jnp.float32),
  )(xp, tp, w1, b1, w2, b2)


def _call_sage(relu, h, msum, deg, ws, wn, b):
  return pl.pallas_call(
      functools.partial(_sage_body, relu),
      grid=(2, _GRID),
      in_specs=[
          pl.BlockSpec((_R, HALF), lambda j, i: (i, 0)),
          pl.BlockSpec((_R, HALF), lambda j, i: (_GRID + i, 0)),
          pl.BlockSpec((_R, HALF), lambda j, i: (i, 0)),
          pl.BlockSpec((_R, HALF), lambda j, i: (_GRID + i, 0)),
          pl.BlockSpec((_R, 1), lambda j, i: (i, 0)),
          pl.BlockSpec((1, 64, HALF), lambda j, i: (j, 0, 0)),
          pl.BlockSpec((1, 64, HALF), lambda j, i: (j, 0, 0)),
          pl.BlockSpec((1, 1, HALF), lambda j, i: (j, 0, 0)),
      ],
      out_specs=pl.BlockSpec((_R, HALF), lambda j, i: (j * _GRID + i, 0)),
      out_shape=jax.ShapeDtypeStruct((2 * N_PAD, HALF), jnp.float32),
  )(h, h, msum, msum, deg, ws, wn, b)


def _call_final(h, msum, deg, ws, wn, b, cws, rws):
  in_specs = [
      pl.BlockSpec((_R, HALF), lambda i: (i, 0)),
      pl.BlockSpec((_R, HALF), lambda i: (_GRID + i, 0)),
      pl.BlockSpec((_R, HALF), lambda i: (i, 0)),
      pl.BlockSpec((_R, HALF), lambda i: (_GRID + i, 0)),
      pl.BlockSpec((_R, 1), lambda i: (i, 0)),
      _full((64, 64)), _full((64, 64)), _full((1, 64)),
  ]
  args = [h, h, msum, msum, deg, ws, wn, b]
  for (w, bb) in cws:
    in_specs += [_full(w.shape), _full(bb.shape)]
    args += [w, bb]
  for (w, bb) in rws:
    in_specs += [_full(w.shape), _full(bb.shape)]
    args += [w, bb]
  return pl.pallas_call(
      _final_body,
      grid=(_GRID,),
      in_specs=in_specs,
      out_specs=[pl.BlockSpec((_R, 8), lambda i: (i, 0)),
                 pl.BlockSpec((_R, 1), lambda i: (i, 0))],
      out_shape=[jax.ShapeDtypeStruct((N_NODES, 8), jnp.float32),
                 jax.ShapeDtypeStruct((N_NODES, 1), jnp.float32)],
  )(*args)


def kernel(feats, params, ntypes, edge_index, dim_list):
  n = feats.shape[0]
  e = edge_index.shape[1]

  tp = ntypes.astype(jnp.int32)[:, None]

  src = edge_index[0].astype(jnp.int32).reshape(e // GROUP, GROUP)
  dst = edge_index[1].astype(jnp.int32).reshape(e // GROUP, GROUP)
  npad_g = E_PAD // GROUP - e // GROUP
  ar = jnp.arange(npad_g * GROUP, dtype=jnp.int32)
  pad_src = (ar % n).reshape(npad_g, GROUP)
  pad_dst = (n + (ar % 128)).reshape(npad_g, GROUP)
  srcp = jnp.concatenate([src, pad_src], axis=0)
  dstp = jnp.concatenate([dst, pad_dst], axis=0)

  def halves(w):
    return jnp.stack([w[..., :HALF], w[..., HALF:]], axis=0)

  pj = params["proj"]
  w1 = jnp.concatenate([pj[t]["l1"]["W"] for t in range(3)], axis=1)
  b1 = jnp.concatenate([pj[t]["l1"]["b"] for t in range(3)])[None, :]
  w2 = halves(jnp.concatenate([pj[t]["l2"]["W"] for t in range(3)], axis=0))
  b2 = halves(jnp.stack([pj[t]["l2"]["b"] for t in range(3)], axis=0))
  sg = params["sage"]
  cws = [(p["W"], p["b"][None, :]) for p in params["cmlp"]]
  rws = [(p["W"], p["b"][None, :]) for p in params["reg"]]

  segsum_deg = _make_segsum(True)
  segsum = _make_segsum(False)

  h0 = _call_proj(feats[:, :16], tp, w1, b1, w2, b2)
  msum0, deg2 = segsum_deg(h0, srcp, dstp)
  deg = deg2[:, None]
  h1 = _call_sage(True, h0, msum0, deg,
                  halves(sg[0]["Wself"]), halves(sg[0]["Wneigh"]),
                  halves(sg[0]["b"][None, :]))
  msum1 = segsum(h1, srcp, dstp)
  if isinstance(msum1, (tuple, list)):
    msum1 = msum1[0]
  cc, rr = _call_final(h1, msum1, deg,
                       sg[1]["Wself"], sg[1]["Wneigh"], sg[1]["b"][None, :],
                       cws, rws)
  return cc, rr
